# baseline (speedup 1.0000x reference)
import jax
import jax.numpy as jnp
import numpy as np
from jax import lax
from jax.experimental import pallas as pl
from jax.experimental.pallas import tpu as pltpu
from jax.experimental.pallas import tpu_sc as plsc

H = W = 512
HW = H * W
B = 4
NC = 2
NS = 16
NW = NC * NS
PPW = HW // NW
SLAB = HW // NS
CH = PPW // 16
MAGIC = np.float32(12582912.0)
BIG = np.float32(H + 1)
INF = np.float32(np.inf)
XMAX = np.float32(W - 1)
YMAX = np.float32(H - 1)


NBLK = 8
BLK = PPW // NBLK
BCH = BLK // 16
HBM_BLOCKS = ()


def _body(mx_hbm, my_hbm, df_hbm, out_hbm,
          shared_x, shared_y, idx_v, gath_x, gath_y,
          dest_x, dest_y, ox_v, oy_v, df_v, sem_x, sem_y):
  cid = lax.axis_index("c")
  sid = lax.axis_index("s")
  base = (cid * NS + sid) * PPW

  lane = lax.iota(jnp.int32, 16)
  pltpu.sync_copy(df_hbm, df_v)
  dfvec = df_v[...]

  for b in range(B):
    n_b = dfvec[b]
    hx_plane = mx_hbm.at[pl.ds(b * HW, HW)]
    hy_plane = my_hbm.at[pl.ds(b * HW, HW)]

    def issue_blk(k):
      ds = pl.ds(k * BLK, BLK)
      sx = hx_plane if k in HBM_BLOCKS else shared_x
      sy = hy_plane if k in HBM_BLOCKS else shared_y
      pltpu.async_copy(sx.at[idx_v.at[ds]], gath_x.at[ds], sem_x.at[k])
      pltpu.async_copy(sy.at[idx_v.at[ds]], gath_y.at[ds], sem_y.at[k])

    def wait_blk(k):
      ds = pl.ds(k * BLK, BLK)
      sx = hx_plane if k in HBM_BLOCKS else shared_x
      sy = hy_plane if k in HBM_BLOCKS else shared_y
      pltpu.make_async_copy(sx.at[idx_v.at[ds]], gath_x.at[ds],
                            sem_x.at[k]).wait()
      pltpu.make_async_copy(sy.at[idx_v.at[ds]], gath_y.at[ds],
                            sem_y.at[k]).wait()

    plsc.subcore_barrier()
    pltpu.sync_copy(mx_hbm.at[pl.ds(b * HW + sid * SLAB, SLAB)],
                    shared_x.at[pl.ds(sid * SLAB, SLAB)])
    pltpu.sync_copy(my_hbm.at[pl.ds(b * HW + sid * SLAB, SLAB)],
                    shared_y.at[pl.ds(sid * SLAB, SLAB)])
    plsc.subcore_barrier()

    def init_chunk(c, _):
      off = c * 16
      lin = base + off + lane
      dest_x[pl.ds(off, 16)] = (lin & (W - 1)).astype(jnp.float32)
      dest_y[pl.ds(off, 16)] = (lin >> 9).astype(jnp.float32)
      idx_v[pl.ds(off, 16)] = lin
      return 0
    lax.fori_loop(0, CH, init_chunk, 0, unroll=4)

    for k in range(NBLK):
      issue_blk(k)

    def step(s, _):
      for k in range(NBLK):
        wait_blk(k)

        def upd(c, _):
          off = k * BLK + c * 16
          ndx = dest_x[pl.ds(off, 16)] + gath_x[pl.ds(off, 16)]
          ndy = dest_y[pl.ds(off, 16)] + gath_y[pl.ds(off, 16)]
          oob = (ndx > XMAX) | (ndx < 0.0) | (ndy > YMAX) | (ndy < 0.0)
          dx2 = jnp.where(oob, INF, ndx)
          dy2 = jnp.where(oob, INF, ndy)
          fx = jnp.minimum(jnp.maximum(dx2, 0.0), XMAX)
          fy = jnp.minimum(jnp.maximum(dy2, 0.0), YMAX)
          xi = ((fx + MAGIC) - MAGIC).astype(jnp.int32)
          yi = ((fy + MAGIC) - MAGIC).astype(jnp.int32)
          dest_x[pl.ds(off, 16)] = dx2
          dest_y[pl.ds(off, 16)] = dy2
          idx_v[pl.ds(off, 16)] = (yi << 9) | xi
          return 0
        lax.fori_loop(0, BCH, upd, 0, unroll=4)
        issue_blk(k)
      return 0
    lax.fori_loop(0, n_b, step, 0)

    for k in range(NBLK):
      wait_blk(k)

    def fin(c, _):
      off = c * 16
      lin = base + off + lane
      cxv = (lin & (W - 1)).astype(jnp.float32)
      cyv = (lin >> 9).astype(jnp.float32)
      dxv = dest_x[pl.ds(off, 16)]
      dyv = dest_y[pl.ds(off, 16)]
      iv = dxv == INF
      ox_v[pl.ds(off, 16)] = jnp.where(iv, BIG, dxv - cxv)
      oy_v[pl.ds(off, 16)] = jnp.where(iv, BIG, dyv - cyv)
      return 0
    lax.fori_loop(0, CH, fin, 0, unroll=4)

    pltpu.sync_copy(ox_v, out_hbm.at[b, 0, pl.ds(base, PPW)])
    pltpu.sync_copy(oy_v, out_hbm.at[b, 1, pl.ds(base, PPW)])


@jax.jit
def kernel(motion, destination_frame):
  mx = motion[:, 0].reshape(B * HW)
  my = motion[:, 1].reshape(B * HW)
  df_pad = jnp.zeros((16,), jnp.int32).at[:B].set(destination_frame)

  mesh = plsc.VectorSubcoreMesh(core_axis_name="c", subcore_axis_name="s")
  run = pl.kernel(
      _body,
      out_type=jax.ShapeDtypeStruct((B, 2, HW), jnp.float32),
      mesh=mesh,
      scratch_types=[
          pltpu.VMEM_SHARED((HW,), jnp.float32),
          pltpu.VMEM_SHARED((HW,), jnp.float32),
          pltpu.VMEM((PPW,), jnp.int32),
          pltpu.VMEM((PPW,), jnp.float32),
          pltpu.VMEM((PPW,), jnp.float32),
          pltpu.VMEM((PPW,), jnp.float32),
          pltpu.VMEM((PPW,), jnp.float32),
          pltpu.VMEM((PPW,), jnp.float32),
          pltpu.VMEM((PPW,), jnp.float32),
          pltpu.VMEM((16,), jnp.int32),
          pltpu.SemaphoreType.DMA((NBLK,)),
          pltpu.SemaphoreType.DMA((NBLK,)),
      ],
  )
  out = run(mx, my, df_pad)
  return out.reshape(B, 2, H, W)


# speedup vs baseline: 1.0153x; 1.0153x over previous
import jax
import jax.numpy as jnp
import numpy as np
from jax import lax
from jax.experimental import pallas as pl
from jax.experimental.pallas import tpu as pltpu
from jax.experimental.pallas import tpu_sc as plsc

H = W = 512
HW = H * W
B = 4
NC = 2
NS = 16
NW = NC * NS
PPW = HW // NW
SLAB = HW // NS
CH = PPW // 16
MAGIC = np.float32(12582912.0)
BIG = np.float32(H + 1)
INF = np.float32(np.inf)
XMAX = np.float32(W - 1)
YMAX = np.float32(H - 1)


NBLK = 4
BLK = PPW // NBLK
BCH = BLK // 16
HBM_BLOCKS = ()


def _body(mx_hbm, my_hbm, df_hbm, out_hbm,
          shared_x, shared_y, idx_v, gath_x, gath_y,
          dest_x, dest_y, ox_v, oy_v, df_v, sem_x, sem_y):
  cid = lax.axis_index("c")
  sid = lax.axis_index("s")
  base = (cid * NS + sid) * PPW

  lane = lax.iota(jnp.int32, 16)
  pltpu.sync_copy(df_hbm, df_v)
  dfvec = df_v[...]

  for b in range(B):
    n_b = dfvec[b]
    hx_plane = mx_hbm.at[pl.ds(b * HW, HW)]
    hy_plane = my_hbm.at[pl.ds(b * HW, HW)]

    def issue_blk(k):
      ds = pl.ds(k * BLK, BLK)
      sx = hx_plane if k in HBM_BLOCKS else shared_x
      sy = hy_plane if k in HBM_BLOCKS else shared_y
      pltpu.async_copy(sx.at[idx_v.at[ds]], gath_x.at[ds], sem_x.at[k])
      pltpu.async_copy(sy.at[idx_v.at[ds]], gath_y.at[ds], sem_y.at[k])

    def wait_blk(k):
      ds = pl.ds(k * BLK, BLK)
      sx = hx_plane if k in HBM_BLOCKS else shared_x
      sy = hy_plane if k in HBM_BLOCKS else shared_y
      pltpu.make_async_copy(sx.at[idx_v.at[ds]], gath_x.at[ds],
                            sem_x.at[k]).wait()
      pltpu.make_async_copy(sy.at[idx_v.at[ds]], gath_y.at[ds],
                            sem_y.at[k]).wait()

    plsc.subcore_barrier()
    pltpu.sync_copy(mx_hbm.at[pl.ds(b * HW + sid * SLAB, SLAB)],
                    shared_x.at[pl.ds(sid * SLAB, SLAB)])
    pltpu.sync_copy(my_hbm.at[pl.ds(b * HW + sid * SLAB, SLAB)],
                    shared_y.at[pl.ds(sid * SLAB, SLAB)])
    plsc.subcore_barrier()

    def init_chunk(c, _):
      off = c * 16
      lin = base + off + lane
      dest_x[pl.ds(off, 16)] = (lin & (W - 1)).astype(jnp.float32)
      dest_y[pl.ds(off, 16)] = (lin >> 9).astype(jnp.float32)
      return 0
    lax.fori_loop(0, CH, init_chunk, 0, unroll=4)

    for k in range(NBLK):
      ds = pl.ds(k * BLK, BLK)
      src = pl.ds(base + k * BLK, BLK)
      pltpu.async_copy(shared_x.at[src], gath_x.at[ds], sem_x.at[k])
      pltpu.async_copy(shared_y.at[src], gath_y.at[ds], sem_y.at[k])

    def step(s, _):
      for k in range(NBLK):
        wait_blk(k)

        def upd(c, _):
          off = k * BLK + c * 16
          ndx = dest_x[pl.ds(off, 16)] + gath_x[pl.ds(off, 16)]
          ndy = dest_y[pl.ds(off, 16)] + gath_y[pl.ds(off, 16)]
          oob = (ndx > XMAX) | (ndx < 0.0) | (ndy > YMAX) | (ndy < 0.0)
          dx2 = jnp.where(oob, INF, ndx)
          dy2 = jnp.where(oob, INF, ndy)
          fx = jnp.minimum(jnp.maximum(dx2, 0.0), XMAX)
          fy = jnp.minimum(jnp.maximum(dy2, 0.0), YMAX)
          xi = ((fx + MAGIC) - MAGIC).astype(jnp.int32)
          yi = ((fy + MAGIC) - MAGIC).astype(jnp.int32)
          dest_x[pl.ds(off, 16)] = dx2
          dest_y[pl.ds(off, 16)] = dy2
          idx_v[pl.ds(off, 16)] = (yi << 9) | xi
          return 0
        lax.fori_loop(0, BCH, upd, 0, unroll=4)
        issue_blk(k)
      return 0
    lax.fori_loop(0, n_b, step, 0)

    for k in range(NBLK):
      wait_blk(k)

    def fin(c, _):
      off = c * 16
      lin = base + off + lane
      cxv = (lin & (W - 1)).astype(jnp.float32)
      cyv = (lin >> 9).astype(jnp.float32)
      dxv = dest_x[pl.ds(off, 16)]
      dyv = dest_y[pl.ds(off, 16)]
      iv = dxv == INF
      ox_v[pl.ds(off, 16)] = jnp.where(iv, BIG, dxv - cxv)
      oy_v[pl.ds(off, 16)] = jnp.where(iv, BIG, dyv - cyv)
      return 0
    lax.fori_loop(0, CH, fin, 0, unroll=4)

    pltpu.sync_copy(ox_v, out_hbm.at[b, 0, pl.ds(base, PPW)])
    pltpu.sync_copy(oy_v, out_hbm.at[b, 1, pl.ds(base, PPW)])


@jax.jit
def kernel(motion, destination_frame):
  mx = motion[:, 0].reshape(B * HW)
  my = motion[:, 1].reshape(B * HW)
  df_pad = jnp.zeros((16,), jnp.int32).at[:B].set(destination_frame)

  mesh = plsc.VectorSubcoreMesh(core_axis_name="c", subcore_axis_name="s")
  run = pl.kernel(
      _body,
      out_type=jax.ShapeDtypeStruct((B, 2, HW), jnp.float32),
      mesh=mesh,
      scratch_types=[
          pltpu.VMEM_SHARED((HW,), jnp.float32),
          pltpu.VMEM_SHARED((HW,), jnp.float32),
          pltpu.VMEM((PPW,), jnp.int32),
          pltpu.VMEM((PPW,), jnp.float32),
          pltpu.VMEM((PPW,), jnp.float32),
          pltpu.VMEM((PPW,), jnp.float32),
          pltpu.VMEM((PPW,), jnp.float32),
          pltpu.VMEM((PPW,), jnp.float32),
          pltpu.VMEM((PPW,), jnp.float32),
          pltpu.VMEM((16,), jnp.int32),
          pltpu.SemaphoreType.DMA((NBLK,)),
          pltpu.SemaphoreType.DMA((NBLK,)),
      ],
  )
  out = run(mx, my, df_pad)
  return out.reshape(B, 2, H, W)


# speedup vs baseline: 1.0256x; 1.0101x over previous
import jax
import jax.numpy as jnp
import numpy as np
from jax import lax
from jax.experimental import pallas as pl
from jax.experimental.pallas import tpu as pltpu
from jax.experimental.pallas import tpu_sc as plsc

H = W = 512
HW = H * W
B = 4
NC = 2
NS = 16
NW = NC * NS
PPW = HW // NW
SLAB = HW // NS
CH = PPW // 16
MAGIC = np.float32(12582912.0)
BIG = np.float32(H + 1)
INF = np.float32(np.inf)
XMAX = np.float32(W - 1)
YMAX = np.float32(H - 1)


NBLK = 4
BLK = PPW // NBLK
BCH = BLK // 16
HBM_BLOCKS = ()


def _body(mx_hbm, my_hbm, df_hbm, out_hbm,
          shared_x, shared_y, idx_v, gath_x, gath_y,
          dest_x, dest_y, ox_v, oy_v, df_v, sem_x, sem_y):
  cid = lax.axis_index("c")
  sid = lax.axis_index("s")
  base = (cid * NS + sid) * PPW

  lane = lax.iota(jnp.int32, 16)
  pltpu.sync_copy(df_hbm, df_v)
  dfvec = df_v[...]

  for b in range(B):
    n_b = dfvec[b]
    hx_plane = mx_hbm.at[pl.ds(b * HW, HW)]
    hy_plane = my_hbm.at[pl.ds(b * HW, HW)]

    def issue_blk(k):
      ds = pl.ds(k * BLK, BLK)
      sx = hx_plane if k in HBM_BLOCKS else shared_x
      sy = hy_plane if k in HBM_BLOCKS else shared_y
      pltpu.async_copy(sx.at[idx_v.at[ds]], gath_x.at[ds], sem_x.at[k])
      pltpu.async_copy(sy.at[idx_v.at[ds]], gath_y.at[ds], sem_y.at[k])

    def wait_blk(k):
      ds = pl.ds(k * BLK, BLK)
      sx = hx_plane if k in HBM_BLOCKS else shared_x
      sy = hy_plane if k in HBM_BLOCKS else shared_y
      pltpu.make_async_copy(sx.at[idx_v.at[ds]], gath_x.at[ds],
                            sem_x.at[k]).wait()
      pltpu.make_async_copy(sy.at[idx_v.at[ds]], gath_y.at[ds],
                            sem_y.at[k]).wait()

    plsc.subcore_barrier()
    pltpu.sync_copy(mx_hbm.at[pl.ds(b * HW + sid * SLAB, SLAB)],
                    shared_x.at[pl.ds(sid * SLAB, SLAB)])
    pltpu.sync_copy(my_hbm.at[pl.ds(b * HW + sid * SLAB, SLAB)],
                    shared_y.at[pl.ds(sid * SLAB, SLAB)])
    plsc.subcore_barrier()

    def init_chunk(c, _):
      off = c * 16
      lin = base + off + lane
      dest_x[pl.ds(off, 16)] = (lin & (W - 1)).astype(jnp.float32)
      dest_y[pl.ds(off, 16)] = (lin >> 9).astype(jnp.float32)
      return 0
    lax.fori_loop(0, CH, init_chunk, 0, unroll=4)

    @pl.when(n_b > 0)
    def _():
      for k in range(NBLK):
        ds = pl.ds(k * BLK, BLK)
        src = pl.ds(base + k * BLK, BLK)
        pltpu.async_copy(shared_x.at[src], gath_x.at[ds], sem_x.at[k])
        pltpu.async_copy(shared_y.at[src], gath_y.at[ds], sem_y.at[k])

    def step(s, _):
      for k in range(NBLK):
        wait_blk(k)

        def upd(c, _):
          off = k * BLK + c * 16
          ndx = dest_x[pl.ds(off, 16)] + gath_x[pl.ds(off, 16)]
          ndy = dest_y[pl.ds(off, 16)] + gath_y[pl.ds(off, 16)]
          oob = (ndx > XMAX) | (ndx < 0.0) | (ndy > YMAX) | (ndy < 0.0)
          dx2 = jnp.where(oob, INF, ndx)
          dy2 = jnp.where(oob, INF, ndy)
          fx = jnp.minimum(jnp.maximum(dx2, 0.0), XMAX)
          fy = jnp.minimum(jnp.maximum(dy2, 0.0), YMAX)
          xi = ((fx + MAGIC) - MAGIC).astype(jnp.int32)
          yi = ((fy + MAGIC) - MAGIC).astype(jnp.int32)
          dest_x[pl.ds(off, 16)] = dx2
          dest_y[pl.ds(off, 16)] = dy2
          idx_v[pl.ds(off, 16)] = (yi << 9) | xi
          return 0
        lax.fori_loop(0, BCH, upd, 0, unroll=4)

        @pl.when(s < n_b - 1)
        def _():
          issue_blk(k)
      return 0
    lax.fori_loop(0, n_b, step, 0)

    def fin(c, _):
      off = c * 16
      lin = base + off + lane
      cxv = (lin & (W - 1)).astype(jnp.float32)
      cyv = (lin >> 9).astype(jnp.float32)
      dxv = dest_x[pl.ds(off, 16)]
      dyv = dest_y[pl.ds(off, 16)]
      iv = dxv == INF
      ox_v[pl.ds(off, 16)] = jnp.where(iv, BIG, dxv - cxv)
      oy_v[pl.ds(off, 16)] = jnp.where(iv, BIG, dyv - cyv)
      return 0
    lax.fori_loop(0, CH, fin, 0, unroll=4)

    pltpu.sync_copy(ox_v, out_hbm.at[b, 0, pl.ds(base, PPW)])
    pltpu.sync_copy(oy_v, out_hbm.at[b, 1, pl.ds(base, PPW)])


@jax.jit
def kernel(motion, destination_frame):
  mx = motion[:, 0].reshape(B * HW)
  my = motion[:, 1].reshape(B * HW)
  df_pad = jnp.zeros((16,), jnp.int32).at[:B].set(destination_frame)

  mesh = plsc.VectorSubcoreMesh(core_axis_name="c", subcore_axis_name="s")
  run = pl.kernel(
      _body,
      out_type=jax.ShapeDtypeStruct((B, 2, HW), jnp.float32),
      mesh=mesh,
      scratch_types=[
          pltpu.VMEM_SHARED((HW,), jnp.float32),
          pltpu.VMEM_SHARED((HW,), jnp.float32),
          pltpu.VMEM((PPW,), jnp.int32),
          pltpu.VMEM((PPW,), jnp.float32),
          pltpu.VMEM((PPW,), jnp.float32),
          pltpu.VMEM((PPW,), jnp.float32),
          pltpu.VMEM((PPW,), jnp.float32),
          pltpu.VMEM((PPW,), jnp.float32),
          pltpu.VMEM((PPW,), jnp.float32),
          pltpu.VMEM((16,), jnp.int32),
          pltpu.SemaphoreType.DMA((NBLK,)),
          pltpu.SemaphoreType.DMA((NBLK,)),
      ],
  )
  out = run(mx, my, df_pad)
  return out.reshape(B, 2, H, W)


# speedup vs baseline: 1.0471x; 1.0210x over previous
import jax
import jax.numpy as jnp
import numpy as np
from jax import lax
from jax.experimental import pallas as pl
from jax.experimental.pallas import tpu as pltpu
from jax.experimental.pallas import tpu_sc as plsc

H = W = 512
HW = H * W
B = 4
NC = 2
NS = 16
NW = NC * NS
PPW = HW // NW
SLAB = HW // NS
CH = PPW // 16
MAGIC = np.float32(12582912.0)
BIG = np.float32(H + 1)
INF = np.float32(np.inf)
XMAX = np.float32(W - 1)
YMAX = np.float32(H - 1)


NBLK = 4
BLK = PPW // NBLK
BCH = BLK // 16


def _body(mx_hbm, my_hbm, df_hbm, out_hbm,
          shared_x0, shared_y0, shared_x1, shared_y1,
          idx_v, gath_x, gath_y,
          dest_x, dest_y, ox_v, oy_v, df_v,
          sem_x, sem_y, sem_stx, sem_sty):
  cid = lax.axis_index("c")
  sid = lax.axis_index("s")
  base = (cid * NS + sid) * PPW

  lane = lax.iota(jnp.int32, 16)
  pltpu.sync_copy(df_hbm, df_v)
  dfvec = df_v[...]

  sx_buf = (shared_x0, shared_x1)
  sy_buf = (shared_y0, shared_y1)

  def stage_copies(b):
    p = b & 1
    src = pl.ds(b * HW + sid * SLAB, SLAB)
    dst = pl.ds(sid * SLAB, SLAB)
    return (
        pltpu.make_async_copy(mx_hbm.at[src], sx_buf[p].at[dst], sem_stx.at[p]),
        pltpu.make_async_copy(my_hbm.at[src], sy_buf[p].at[dst], sem_sty.at[p]),
    )

  for c in stage_copies(0):
    c.start()

  for b in range(B):
    n_b = dfvec[b]
    shared_x = sx_buf[b & 1]
    shared_y = sy_buf[b & 1]

    def issue_blk(k):
      ds = pl.ds(k * BLK, BLK)
      pltpu.async_copy(shared_x.at[idx_v.at[ds]], gath_x.at[ds], sem_x.at[k])
      pltpu.async_copy(shared_y.at[idx_v.at[ds]], gath_y.at[ds], sem_y.at[k])

    def wait_blk(k):
      ds = pl.ds(k * BLK, BLK)
      pltpu.make_async_copy(shared_x.at[idx_v.at[ds]], gath_x.at[ds],
                            sem_x.at[k]).wait()
      pltpu.make_async_copy(shared_y.at[idx_v.at[ds]], gath_y.at[ds],
                            sem_y.at[k]).wait()

    for c in stage_copies(b):
      c.wait()
    plsc.subcore_barrier()
    if b + 1 < B:
      for c in stage_copies(b + 1):
        c.start()

    def init_chunk(c, _):
      off = c * 16
      lin = base + off + lane
      dest_x[pl.ds(off, 16)] = (lin & (W - 1)).astype(jnp.float32)
      dest_y[pl.ds(off, 16)] = (lin >> 9).astype(jnp.float32)
      return 0
    lax.fori_loop(0, CH, init_chunk, 0, unroll=4)

    @pl.when(n_b > 0)
    def _():
      for k in range(NBLK):
        ds = pl.ds(k * BLK, BLK)
        src = pl.ds(base + k * BLK, BLK)
        pltpu.async_copy(shared_x.at[src], gath_x.at[ds], sem_x.at[k])
        pltpu.async_copy(shared_y.at[src], gath_y.at[ds], sem_y.at[k])

    def step(s, _):
      for k in range(NBLK):
        wait_blk(k)

        def upd(c, _):
          off = k * BLK + c * 16
          ndx = dest_x[pl.ds(off, 16)] + gath_x[pl.ds(off, 16)]
          ndy = dest_y[pl.ds(off, 16)] + gath_y[pl.ds(off, 16)]
          oob = (ndx > XMAX) | (ndx < 0.0) | (ndy > YMAX) | (ndy < 0.0)
          dx2 = jnp.where(oob, INF, ndx)
          dy2 = jnp.where(oob, INF, ndy)
          fx = jnp.minimum(jnp.maximum(dx2, 0.0), XMAX)
          fy = jnp.minimum(jnp.maximum(dy2, 0.0), YMAX)
          xi = ((fx + MAGIC) - MAGIC).astype(jnp.int32)
          yi = ((fy + MAGIC) - MAGIC).astype(jnp.int32)
          dest_x[pl.ds(off, 16)] = dx2
          dest_y[pl.ds(off, 16)] = dy2
          idx_v[pl.ds(off, 16)] = (yi << 9) | xi
          return 0
        lax.fori_loop(0, BCH, upd, 0, unroll=4)

        @pl.when(s < n_b - 1)
        def _():
          issue_blk(k)
      return 0
    lax.fori_loop(0, n_b, step, 0)

    def fin(c, _):
      off = c * 16
      lin = base + off + lane
      cxv = (lin & (W - 1)).astype(jnp.float32)
      cyv = (lin >> 9).astype(jnp.float32)
      dxv = dest_x[pl.ds(off, 16)]
      dyv = dest_y[pl.ds(off, 16)]
      iv = dxv == INF
      ox_v[pl.ds(off, 16)] = jnp.where(iv, BIG, dxv - cxv)
      oy_v[pl.ds(off, 16)] = jnp.where(iv, BIG, dyv - cyv)
      return 0
    lax.fori_loop(0, CH, fin, 0, unroll=4)

    pltpu.sync_copy(ox_v, out_hbm.at[b, 0, pl.ds(base, PPW)])
    pltpu.sync_copy(oy_v, out_hbm.at[b, 1, pl.ds(base, PPW)])


@jax.jit
def kernel(motion, destination_frame):
  mx = motion[:, 0].reshape(B * HW)
  my = motion[:, 1].reshape(B * HW)
  df_pad = jnp.zeros((16,), jnp.int32).at[:B].set(destination_frame)

  mesh = plsc.VectorSubcoreMesh(core_axis_name="c", subcore_axis_name="s")
  run = pl.kernel(
      _body,
      out_type=jax.ShapeDtypeStruct((B, 2, HW), jnp.float32),
      mesh=mesh,
      scratch_types=[
          pltpu.VMEM_SHARED((HW,), jnp.float32),
          pltpu.VMEM_SHARED((HW,), jnp.float32),
          pltpu.VMEM_SHARED((HW,), jnp.float32),
          pltpu.VMEM_SHARED((HW,), jnp.float32),
          pltpu.VMEM((PPW,), jnp.int32),
          pltpu.VMEM((PPW,), jnp.float32),
          pltpu.VMEM((PPW,), jnp.float32),
          pltpu.VMEM((PPW,), jnp.float32),
          pltpu.VMEM((PPW,), jnp.float32),
          pltpu.VMEM((PPW,), jnp.float32),
          pltpu.VMEM((PPW,), jnp.float32),
          pltpu.VMEM((16,), jnp.int32),
          pltpu.SemaphoreType.DMA((NBLK,)),
          pltpu.SemaphoreType.DMA((NBLK,)),
          pltpu.SemaphoreType.DMA((2,)),
          pltpu.SemaphoreType.DMA((2,)),
      ],
  )
  out = run(mx, my, df_pad)
  return out.reshape(B, 2, H, W)


# speedup vs baseline: 1.0552x; 1.0077x over previous
import jax
import jax.numpy as jnp
import numpy as np
from jax import lax
from jax.experimental import pallas as pl
from jax.experimental.pallas import tpu as pltpu
from jax.experimental.pallas import tpu_sc as plsc

H = W = 512
HW = H * W
B = 4
NC = 2
NS = 16
NW = NC * NS
PPW = HW // NW
SLAB = HW // NS
CH = PPW // 16
MAGIC = np.float32(12582912.0)
BIG = np.float32(H + 1)
INF = np.float32(np.inf)
XMAX = np.float32(W - 1)
YMAX = np.float32(H - 1)


NBLK = 4
BLK = PPW // NBLK
BCH = BLK // 16


def _body(mx_hbm, my_hbm, df_hbm, out_hbm,
          shared_x0, shared_y0, shared_x1, shared_y1,
          idx_v, gath_x, gath_y,
          dest_x, dest_y, ox_v, oy_v, df_v,
          sem_x, sem_y, sem_stx, sem_sty, sem_ox, sem_oy):
  cid = lax.axis_index("c")
  sid = lax.axis_index("s")
  base = (cid * NS + sid) * PPW

  lane = lax.iota(jnp.int32, 16)
  pltpu.sync_copy(df_hbm, df_v)
  dfvec = df_v[...]

  sx_buf = (shared_x0, shared_x1)
  sy_buf = (shared_y0, shared_y1)
  def out_copies(b):
    return (
        pltpu.make_async_copy(ox_v, out_hbm.at[b, 0, pl.ds(base, PPW)],
                              sem_ox),
        pltpu.make_async_copy(oy_v, out_hbm.at[b, 1, pl.ds(base, PPW)],
                              sem_oy),
    )

  def stage_copies(b):
    p = b & 1
    src = pl.ds(b * HW + sid * SLAB, SLAB)
    dst = pl.ds(sid * SLAB, SLAB)
    return (
        pltpu.make_async_copy(mx_hbm.at[src], sx_buf[p].at[dst], sem_stx.at[p]),
        pltpu.make_async_copy(my_hbm.at[src], sy_buf[p].at[dst], sem_sty.at[p]),
    )

  for c in stage_copies(0):
    c.start()

  for b in range(B):
    n_b = dfvec[b]
    shared_x = sx_buf[b & 1]
    shared_y = sy_buf[b & 1]

    def issue_blk(k):
      ds = pl.ds(k * BLK, BLK)
      pltpu.async_copy(shared_x.at[idx_v.at[ds]], gath_x.at[ds], sem_x.at[k])
      pltpu.async_copy(shared_y.at[idx_v.at[ds]], gath_y.at[ds], sem_y.at[k])

    def wait_blk(k):
      ds = pl.ds(k * BLK, BLK)
      pltpu.make_async_copy(shared_x.at[idx_v.at[ds]], gath_x.at[ds],
                            sem_x.at[k]).wait()
      pltpu.make_async_copy(shared_y.at[idx_v.at[ds]], gath_y.at[ds],
                            sem_y.at[k]).wait()

    for c in stage_copies(b):
      c.wait()
    plsc.subcore_barrier()
    if b + 1 < B:
      for c in stage_copies(b + 1):
        c.start()

    def init_chunk(c, _):
      off = c * 16
      lin = base + off + lane
      dest_x[pl.ds(off, 16)] = (lin & (W - 1)).astype(jnp.float32)
      dest_y[pl.ds(off, 16)] = (lin >> 9).astype(jnp.float32)
      return 0
    lax.fori_loop(0, CH, init_chunk, 0, unroll=4)

    @pl.when(n_b > 0)
    def _():
      for k in range(NBLK):
        ds = pl.ds(k * BLK, BLK)
        src = pl.ds(base + k * BLK, BLK)
        pltpu.async_copy(shared_x.at[src], gath_x.at[ds], sem_x.at[k])
        pltpu.async_copy(shared_y.at[src], gath_y.at[ds], sem_y.at[k])

    def step(s, _):
      for k in range(NBLK):
        wait_blk(k)

        def upd(c, _):
          off = k * BLK + c * 16
          ndx = dest_x[pl.ds(off, 16)] + gath_x[pl.ds(off, 16)]
          ndy = dest_y[pl.ds(off, 16)] + gath_y[pl.ds(off, 16)]
          oob = (ndx > XMAX) | (ndx < 0.0) | (ndy > YMAX) | (ndy < 0.0)
          dx2 = jnp.where(oob, INF, ndx)
          dy2 = jnp.where(oob, INF, ndy)
          fx = jnp.minimum(jnp.maximum(dx2, 0.0), XMAX)
          fy = jnp.minimum(jnp.maximum(dy2, 0.0), YMAX)
          xi = ((fx + MAGIC) - MAGIC).astype(jnp.int32)
          yi = ((fy + MAGIC) - MAGIC).astype(jnp.int32)
          dest_x[pl.ds(off, 16)] = dx2
          dest_y[pl.ds(off, 16)] = dy2
          idx_v[pl.ds(off, 16)] = (yi << 9) | xi
          return 0
        lax.fori_loop(0, BCH, upd, 0, unroll=4)

        @pl.when(s < n_b - 1)
        def _():
          issue_blk(k)
      return 0
    lax.fori_loop(0, n_b, step, 0)

    if b >= 1:
      for cpy in out_copies(b - 1):
        cpy.wait()

    def fin(c, _):
      off = c * 16
      lin = base + off + lane
      cxv = (lin & (W - 1)).astype(jnp.float32)
      cyv = (lin >> 9).astype(jnp.float32)
      dxv = dest_x[pl.ds(off, 16)]
      dyv = dest_y[pl.ds(off, 16)]
      iv = dxv == INF
      ox_v[pl.ds(off, 16)] = jnp.where(iv, BIG, dxv - cxv)
      oy_v[pl.ds(off, 16)] = jnp.where(iv, BIG, dyv - cyv)
      return 0
    lax.fori_loop(0, CH, fin, 0, unroll=4)

    for cpy in out_copies(b):
      cpy.start()

  for cpy in out_copies(B - 1):
    cpy.wait()


@jax.jit
def kernel(motion, destination_frame):
  mx = motion[:, 0].reshape(B * HW)
  my = motion[:, 1].reshape(B * HW)
  df_pad = jnp.zeros((16,), jnp.int32).at[:B].set(destination_frame)

  mesh = plsc.VectorSubcoreMesh(core_axis_name="c", subcore_axis_name="s")
  run = pl.kernel(
      _body,
      out_type=jax.ShapeDtypeStruct((B, 2, HW), jnp.float32),
      mesh=mesh,
      scratch_types=[
          pltpu.VMEM_SHARED((HW,), jnp.float32),
          pltpu.VMEM_SHARED((HW,), jnp.float32),
          pltpu.VMEM_SHARED((HW,), jnp.float32),
          pltpu.VMEM_SHARED((HW,), jnp.float32),
          pltpu.VMEM((PPW,), jnp.int32),
          pltpu.VMEM((PPW,), jnp.float32),
          pltpu.VMEM((PPW,), jnp.float32),
          pltpu.VMEM((PPW,), jnp.float32),
          pltpu.VMEM((PPW,), jnp.float32),
          pltpu.VMEM((PPW,), jnp.float32),
          pltpu.VMEM((PPW,), jnp.float32),
          pltpu.VMEM((16,), jnp.int32),
          pltpu.SemaphoreType.DMA((NBLK,)),
          pltpu.SemaphoreType.DMA((NBLK,)),
          pltpu.SemaphoreType.DMA((2,)),
          pltpu.SemaphoreType.DMA((2,)),
          pltpu.SemaphoreType.DMA,
          pltpu.SemaphoreType.DMA,
      ],
  )
  out = run(mx, my, df_pad)
  return out.reshape(B, 2, H, W)
